# baseline (speedup 1.0000x reference)
import functools

import jax
import jax.numpy as jnp
from jax import lax
from jax.experimental import pallas as pl
from jax.experimental.pallas import tpu as pltpu
from jax.experimental.pallas import tpu_sc as plsc

B = 16384
N_LAYERS = 8
VOCAB = 100000
DIM = 64
ROWS = N_LAYERS * VOCAB
PAD = 128

NUM_CORES = 2
NUM_SUBCORES = 16
NUM_WORKERS = NUM_CORES * NUM_SUBCORES
TOK_PER_WORKER = B // NUM_WORKERS
CHUNK = 128
NUM_CHUNKS = TOK_PER_WORKER // CHUNK
LANES = 16

STAGE_BV = 12800
STAGE_GRID = -(-VOCAB // STAGE_BV)

_mesh = plsc.VectorSubcoreMesh(core_axis_name="c", subcore_axis_name="s")
_sc_params = pltpu.CompilerParams(
    needs_layout_passes=False, use_tc_tiling_on_sc=True
)


HGRP = N_LAYERS


def _stage_body(i_ref, o_ref):
    x = i_ref[0]
    xt = x.T
    o_ref[0] = jnp.concatenate([xt, xt], axis=1)


_stage = pl.pallas_call(
    _stage_body,
    grid=(HGRP, STAGE_GRID),
    in_specs=[pl.BlockSpec((1, DIM, STAGE_BV), lambda l, i: (l, 0, i))],
    out_specs=pl.BlockSpec((1, STAGE_BV, PAD), lambda l, i: (l, i, 0)),
    out_shape=jax.ShapeDtypeStruct((HGRP, VOCAB, PAD), jnp.float32),
)


def _make_lookup(first):
    def _body(x_hbm, tab_hbm, *rest):
        if first:
            out_hbm, xv, fi, acc, sem = rest
        else:
            partial_hbm, out_hbm, xv, fi, acc, sem = rest
        base = (
            lax.axis_index("s") * NUM_CORES + lax.axis_index("c")
        ) * TOK_PER_WORKER
        zeros = jnp.zeros((LANES,), jnp.float32)

        pltpu.sync_copy(x_hbm.at[:, pl.ds(base, TOK_PER_WORKER)], xv)
        for l in range(HGRP):
            for v in range(TOK_PER_WORKER // LANES):
                sl = pl.ds(v * LANES, LANES)
                fi[l, sl] = xv[l, sl] + l * VOCAB
        for t in range(CHUNK * NUM_CHUNKS):
            for s in range(PAD // LANES):
                acc[t, pl.ds(s * LANES, LANES)] = zeros
        copies = []
        for ci in range(NUM_CHUNKS):
            for l in range(HGRP):
                copies.append(
                    pltpu.async_copy(
                        tab_hbm.at[fi.at[l, pl.ds(ci * CHUNK, CHUNK)]],
                        acc.at[pl.ds(ci * CHUNK, CHUNK)],
                        sem,
                        add=True,
                    )
                )
        for c in copies:
            c.wait()
        pltpu.sync_copy(acc, out_hbm.at[pl.ds(base, TOK_PER_WORKER)])

    return pl.kernel(
        _body,
        out_type=jax.ShapeDtypeStruct((B, PAD), jnp.float32),
        mesh=_mesh,
        compiler_params=_sc_params,
        scratch_types=[
            pltpu.VMEM((HGRP, TOK_PER_WORKER), jnp.int32),
            pltpu.VMEM((HGRP, TOK_PER_WORKER), jnp.int32),
            pltpu.VMEM((TOK_PER_WORKER, PAD), jnp.float32),
            pltpu.SemaphoreType.DMA,
        ],
    )


_lookup = _make_lookup(first=True)


def kernel(x, emb):
    x_t = x.astype(jnp.int32).T
    emb_t = jnp.transpose(emb, (0, 2, 1))
    tab = _stage(emb_t).reshape(ROWS, PAD)
    wide = _lookup(x_t, tab)
    return wide[:, :DIM]


# speedup vs baseline: 1.0050x; 1.0050x over previous
import functools

import jax
import jax.numpy as jnp
from jax import lax
from jax.experimental import pallas as pl
from jax.experimental.pallas import tpu as pltpu
from jax.experimental.pallas import tpu_sc as plsc

B = 16384
N_LAYERS = 8
VOCAB = 100000
DIM = 64
ROWS = N_LAYERS * VOCAB
PAD = 128

NUM_CORES = 2
NUM_SUBCORES = 16
NUM_WORKERS = NUM_CORES * NUM_SUBCORES
TOK_PER_WORKER = B // NUM_WORKERS
CHUNK = 128
NUM_CHUNKS = TOK_PER_WORKER // CHUNK
LANES = 16

STAGE_BV = 12800
STAGE_GRID = -(-VOCAB // STAGE_BV)

_mesh = plsc.VectorSubcoreMesh(core_axis_name="c", subcore_axis_name="s")
_sc_params = pltpu.CompilerParams(
    needs_layout_passes=False, use_tc_tiling_on_sc=True
)


def _stage_body(i_ref, o_ref):
    x = i_ref[0]
    xt = x.T
    o_ref[0] = jnp.concatenate([xt, xt], axis=1)


_stage = pl.pallas_call(
    _stage_body,
    grid=(N_LAYERS, STAGE_GRID),
    in_specs=[pl.BlockSpec((1, DIM, STAGE_BV), lambda l, i: (l, 0, i))],
    out_specs=pl.BlockSpec((1, STAGE_BV, PAD), lambda l, i: (l, i, 0)),
    out_shape=jax.ShapeDtypeStruct((N_LAYERS, VOCAB, PAD), jnp.float32),
)


@functools.partial(
    pl.kernel,
    out_type=jax.ShapeDtypeStruct((B, PAD), jnp.float32),
    mesh=_mesh,
    compiler_params=_sc_params,
    scratch_types=[
        pltpu.VMEM((N_LAYERS, CHUNK), jnp.int32),
        pltpu.VMEM((N_LAYERS, CHUNK), jnp.int32),
        pltpu.VMEM((CHUNK, PAD), jnp.float32),
        pltpu.SemaphoreType.DMA,
    ],
)
def _lookup(x_hbm, tab_hbm, out_hbm, xv, fi, acc, sem):
    base = (
        lax.axis_index("s") * NUM_CORES + lax.axis_index("c")
    ) * TOK_PER_WORKER
    zeros = jnp.zeros((LANES,), jnp.float32)

    @pl.loop(0, NUM_CHUNKS)
    def _chunk(ci):
        tok = base + ci * CHUNK
        pltpu.sync_copy(x_hbm.at[:, pl.ds(tok, CHUNK)], xv)
        for l in range(N_LAYERS):
            for v in range(CHUNK // LANES):
                sl = pl.ds(v * LANES, LANES)
                fi[l, sl] = xv[l, sl] + l * VOCAB
        for t in range(CHUNK):
            for s in range(PAD // LANES):
                acc[t, pl.ds(s * LANES, LANES)] = zeros
        copies = [
            pltpu.async_copy(tab_hbm.at[fi.at[l]], acc, sem, add=True)
            for l in range(N_LAYERS)
        ]
        for c in copies:
            c.wait()
        pltpu.sync_copy(acc, out_hbm.at[pl.ds(tok, CHUNK)])


def kernel(x, emb):
    x_t = x.astype(jnp.int32).T
    emb_t = jnp.transpose(emb, (0, 2, 1))
    tab = _stage(emb_t).reshape(ROWS, PAD)
    wide = _lookup(x_t, tab)
    return wide[:, :DIM]
